# baseline (speedup 1.0000x reference)
import functools

import jax
import jax.numpy as jnp
from jax import lax
from jax.experimental import pallas as pl
from jax.experimental.pallas import tpu as pltpu
from jax.experimental.pallas import tpu_sc as plsc

_NC = 2
_NS = 16
_W = _NC * _NS
_LANES = 16
_CHROWS = 25
_NBUF = 5
_NEG = float("-inf")


def _build_partial(S, N, interpret=False):
    NPW = N // _W
    NCH = NPW // _CHROWS
    NV = S // _LANES
    QUAD = NV // 4
    CH = _CHROWS * S
    assert N % (_W * _CHROWS) == 0 and S % (4 * _LANES) == 0
    assert NCH % _NBUF == 0

    mesh = plsc.VectorSubcoreMesh(
        core_axis_name="c", subcore_axis_name="s",
        num_cores=_NC, num_subcores=_NS)

    @functools.partial(
        pl.kernel,
        out_type=(jax.ShapeDtypeStruct((_W * S,), jnp.float32),
                  jax.ShapeDtypeStruct((_W * S,), jnp.int32)),
        mesh=mesh,
        scratch_types=(
            [pltpu.VMEM((CH,), jnp.float32) for _ in range(_NBUF)] + [
                pltpu.VMEM((S,), jnp.float32),
                pltpu.VMEM((S,), jnp.int32),
            ] + [pltpu.SemaphoreType.DMA for _ in range(_NBUF)]
        ),
        compiler_params=pltpu.CompilerParams(use_tc_tiling_on_sc=False),
        interpret=interpret,
    )
    def run(obj_hbm, pmax_hbm, pidx_hbm, *rest):
        bufs = rest[:_NBUF]
        pmax, pidx = rest[_NBUF], rest[_NBUF + 1]
        sems = rest[_NBUF + 2:]
        w = lax.axis_index("s") * _NC + lax.axis_index("c")
        n0 = w * NPW

        def copy_chunk(ch, par):
            return pltpu.make_async_copy(
                obj_hbm.at[pl.ds((n0 + ch * _CHROWS) * S, CH)],
                bufs[par], sems[par])

        IN_FLIGHT = _NBUF - 1

        def consume(buf, ch):
            def qbody(q, carry, buf=buf):
                base = q * (QUAD * _LANES)
                ms = [pmax[pl.ds(base + u * _LANES, _LANES)]
                      for u in range(QUAD)]
                mis = [pidx[pl.ds(base + u * _LANES, _LANES)]
                       for u in range(QUAD)]

                def body(j, c8, buf=buf):
                    m8, mi8 = c8
                    m8, mi8 = list(m8), list(mi8)
                    nsplat = jnp.broadcast_to(
                        (n0 + ch * _CHROWS + j).astype(jnp.int32), (_LANES,))
                    for u in range(QUAD):
                        v = buf[pl.ds(j * S + base + u * _LANES, _LANES)]
                        cmp = v > m8[u]
                        m8[u] = jnp.where(cmp, v, m8[u])
                        mi8[u] = jnp.where(cmp, nsplat, mi8[u])
                    return tuple(m8), tuple(mi8)

                ms, mis = lax.fori_loop(
                    0, _CHROWS, body, (tuple(ms), tuple(mis)))
                for u in range(QUAD):
                    pmax[pl.ds(base + u * _LANES, _LANES)] = ms[u]
                    pidx[pl.ds(base + u * _LANES, _LANES)] = mis[u]
                return carry

            lax.fori_loop(0, 4, qbody, 0)

        neg = jnp.full((_LANES,), _NEG, jnp.float32)
        zero = jnp.zeros((_LANES,), jnp.int32)
        for u in range(NV):
            pmax[pl.ds(u * _LANES, _LANES)] = neg
            pidx[pl.ds(u * _LANES, _LANES)] = zero

        for b in range(IN_FLIGHT):
            copy_chunk(b, b).start()

        def ring(p, carry):
            for b in range(_NBUF):
                ch = p * _NBUF + b

                @pl.when(ch + IN_FLIGHT < NCH)
                def _(ch=ch, b=b):
                    copy_chunk(ch + IN_FLIGHT, (b + IN_FLIGHT) % _NBUF).start()

                copy_chunk(ch, b).wait()
                consume(bufs[b], ch)
            return carry

        lax.fori_loop(0, NCH // _NBUF, ring, 0)

        pltpu.sync_copy(pmax, pmax_hbm.at[pl.ds(w * S, S)])
        pltpu.sync_copy(pidx, pidx_hbm.at[pl.ds(w * S, S)])

    return run


def _build_merge(S, N, D, interpret=False):
    mesh = plsc.VectorSubcoreMesh(
        core_axis_name="c", subcore_axis_name="s",
        num_cores=_NC, num_subcores=_NS)

    @functools.partial(
        pl.kernel,
        out_type=(jax.ShapeDtypeStruct((D * S,), jnp.float32),
                  jax.ShapeDtypeStruct((S,), jnp.float32)),
        mesh=mesh,
        scratch_types=[
            pltpu.VMEM((_W, _LANES), jnp.float32),
            pltpu.VMEM((_W, _LANES), jnp.int32),
            pltpu.VMEM((D * _LANES,), jnp.int32),
            pltpu.VMEM((D * _LANES,), jnp.float32),
            pltpu.VMEM((_LANES,), jnp.float32),
            pltpu.SemaphoreType.DMA,
            pltpu.SemaphoreType.DMA,
            pltpu.SemaphoreType.DMA,
        ],
        compiler_params=pltpu.CompilerParams(use_tc_tiling_on_sc=False),
        interpret=interpret,
    )
    def run(pmax_hbm, pidx_hbm, xf_hbm, outx_hbm, outs_hbm,
            mgm, mgi, idxq, gbuf, valv, sem0, sem1, gsem):
        w = lax.axis_index("s") * _NC + lax.axis_index("c")
        s0 = w * _LANES

        cps = []
        for k in range(_W):
            cm = pltpu.make_async_copy(
                pmax_hbm.at[pl.ds(k * S + s0, _LANES)], mgm.at[k], sem0)
            ci = pltpu.make_async_copy(
                pidx_hbm.at[pl.ds(k * S + s0, _LANES)], mgi.at[k], sem1)
            cm.start()
            ci.start()
            cps.append((cm, ci))
        for cm, ci in cps:
            cm.wait()
            ci.wait()

        m = jnp.full((_LANES,), _NEG, jnp.float32)
        mi = jnp.zeros((_LANES,), jnp.int32)
        for k in range(_W):
            v = mgm[k]
            vi = mgi[k]
            cmp = v > m
            m = jnp.where(cmp, v, m)
            mi = jnp.where(cmp, vi, mi)

        valv[...] = m
        for c in range(D):
            idxq[pl.ds(c * _LANES, _LANES)] = mi + c * N
        pltpu.async_copy(xf_hbm.at[idxq], gbuf, gsem).wait()
        ocps = []
        for c in range(D):
            cp = pltpu.make_async_copy(
                gbuf.at[pl.ds(c * _LANES, _LANES)],
                outx_hbm.at[pl.ds(c * S + s0, _LANES)], sem0)
            cp.start()
            ocps.append(cp)
        for cp in ocps:
            cp.wait()
        pltpu.sync_copy(valv, outs_hbm.at[pl.ds(s0, _LANES)])

    return run


def kernel(X, samples, num_samples):
    S, N = samples.shape[0], samples.shape[1]
    D = X.shape[-1]
    obj_flat = jnp.transpose(samples, (2, 1, 0)).reshape(-1)
    pmax, pidx = _build_partial(S, N)(obj_flat)
    x_cmaj, score = _build_merge(S, N, D)(pmax, pidx, X.T.reshape(-1))
    return x_cmaj.reshape(D, S).T, score.reshape(S, 1)


# speedup vs baseline: 1.0179x; 1.0179x over previous
import functools

import jax
import jax.numpy as jnp
from jax import lax
from jax.experimental import pallas as pl
from jax.experimental.pallas import tpu as pltpu
from jax.experimental.pallas import tpu_sc as plsc

_NC = 2
_NS = 16
_W = _NC * _NS
_LANES = 16
_CHROWS = 25
_NBUF = 5
_NEG = float("-inf")


def _build_partial(S, N, interpret=False):
    NPW = N // _W
    NCH = NPW // _CHROWS
    NV = S // _LANES
    QUAD = NV // 4
    CH = _CHROWS * S
    assert N % (_W * _CHROWS) == 0 and S % (4 * _LANES) == 0
    assert NCH % _NBUF == 0

    mesh = plsc.VectorSubcoreMesh(
        core_axis_name="c", subcore_axis_name="s",
        num_cores=_NC, num_subcores=_NS)

    @functools.partial(
        pl.kernel,
        out_type=(jax.ShapeDtypeStruct((_W * S,), jnp.float32),
                  jax.ShapeDtypeStruct((_W * S,), jnp.int32)),
        mesh=mesh,
        scratch_types=(
            [pltpu.VMEM((CH,), jnp.float32) for _ in range(_NBUF)] + [
                pltpu.VMEM((S,), jnp.float32),
                pltpu.VMEM((S,), jnp.int32),
            ] + [pltpu.SemaphoreType.DMA for _ in range(_NBUF)]
        ),
        compiler_params=pltpu.CompilerParams(use_tc_tiling_on_sc=False),
        interpret=interpret,
    )
    def run(obj_hbm, pmax_hbm, pidx_hbm, *rest):
        bufs = rest[:_NBUF]
        pmax, pidx = rest[_NBUF], rest[_NBUF + 1]
        sems = rest[_NBUF + 2:]
        w = lax.axis_index("s") * _NC + lax.axis_index("c")
        n0 = w * NPW

        def copy_chunk(ch, par):
            return pltpu.make_async_copy(
                obj_hbm.at[pl.ds((n0 + ch * _CHROWS) * S, CH)],
                bufs[par], sems[par])

        IN_FLIGHT = _NBUF - 1

        def consume(buf, ch):
            for q in range(4):
                ms = [pmax[pl.ds((q * QUAD + u) * _LANES, _LANES)]
                      for u in range(QUAD)]
                mis = [pidx[pl.ds((q * QUAD + u) * _LANES, _LANES)]
                       for u in range(QUAD)]

                def body(j, carry, q=q, buf=buf):
                    m8, mi8 = carry
                    m8, mi8 = list(m8), list(mi8)
                    nsplat = jnp.broadcast_to(
                        (n0 + ch * _CHROWS + j).astype(jnp.int32), (_LANES,))
                    for u in range(QUAD):
                        v = buf[pl.ds(j * S + (q * QUAD + u) * _LANES,
                                      _LANES)]
                        cmp = v > m8[u]
                        m8[u] = jnp.where(cmp, v, m8[u])
                        mi8[u] = jnp.where(cmp, nsplat, mi8[u])
                    return tuple(m8), tuple(mi8)

                ms, mis = lax.fori_loop(
                    0, _CHROWS, body, (tuple(ms), tuple(mis)))
                for u in range(QUAD):
                    pmax[pl.ds((q * QUAD + u) * _LANES, _LANES)] = ms[u]
                    pidx[pl.ds((q * QUAD + u) * _LANES, _LANES)] = mis[u]

        neg = jnp.full((_LANES,), _NEG, jnp.float32)
        zero = jnp.zeros((_LANES,), jnp.int32)
        for u in range(NV):
            pmax[pl.ds(u * _LANES, _LANES)] = neg
            pidx[pl.ds(u * _LANES, _LANES)] = zero

        for b in range(IN_FLIGHT):
            copy_chunk(b, b).start()

        def ring(p, carry):
            for b in range(_NBUF):
                ch = p * _NBUF + b

                @pl.when(ch + IN_FLIGHT < NCH)
                def _(ch=ch, b=b):
                    copy_chunk(ch + IN_FLIGHT, (b + IN_FLIGHT) % _NBUF).start()

                copy_chunk(ch, b).wait()
                consume(bufs[b], ch)
            return carry

        lax.fori_loop(0, NCH // _NBUF, ring, 0)

        pltpu.sync_copy(pmax, pmax_hbm.at[pl.ds(w * S, S)])
        pltpu.sync_copy(pidx, pidx_hbm.at[pl.ds(w * S, S)])

    return run


def _build_merge(S, N, D, interpret=False):
    mesh = plsc.VectorSubcoreMesh(
        core_axis_name="c", subcore_axis_name="s",
        num_cores=_NC, num_subcores=_NS)

    @functools.partial(
        pl.kernel,
        out_type=(jax.ShapeDtypeStruct((D * S,), jnp.float32),
                  jax.ShapeDtypeStruct((S,), jnp.float32)),
        mesh=mesh,
        scratch_types=[
            pltpu.VMEM((_W, _LANES), jnp.float32),
            pltpu.VMEM((_W, _LANES), jnp.int32),
            pltpu.VMEM((D * _LANES,), jnp.int32),
            pltpu.VMEM((D * _LANES,), jnp.float32),
            pltpu.VMEM((_LANES,), jnp.float32),
            pltpu.SemaphoreType.DMA,
            pltpu.SemaphoreType.DMA,
            pltpu.SemaphoreType.DMA,
        ],
        compiler_params=pltpu.CompilerParams(use_tc_tiling_on_sc=False),
        interpret=interpret,
    )
    def run(pmax_hbm, pidx_hbm, xf_hbm, outx_hbm, outs_hbm,
            mgm, mgi, idxq, gbuf, valv, sem0, sem1, gsem):
        w = lax.axis_index("s") * _NC + lax.axis_index("c")
        s0 = w * _LANES

        cps = []
        for k in range(_W):
            cm = pltpu.make_async_copy(
                pmax_hbm.at[pl.ds(k * S + s0, _LANES)], mgm.at[k], sem0)
            ci = pltpu.make_async_copy(
                pidx_hbm.at[pl.ds(k * S + s0, _LANES)], mgi.at[k], sem1)
            cm.start()
            ci.start()
            cps.append((cm, ci))
        for cm, ci in cps:
            cm.wait()
            ci.wait()

        m = jnp.full((_LANES,), _NEG, jnp.float32)
        mi = jnp.zeros((_LANES,), jnp.int32)
        for k in range(_W):
            v = mgm[k]
            vi = mgi[k]
            cmp = v > m
            m = jnp.where(cmp, v, m)
            mi = jnp.where(cmp, vi, mi)

        valv[...] = m
        for c in range(D):
            idxq[pl.ds(c * _LANES, _LANES)] = mi + c * N
        pltpu.async_copy(xf_hbm.at[idxq], gbuf, gsem).wait()
        ocps = []
        for c in range(D):
            cp = pltpu.make_async_copy(
                gbuf.at[pl.ds(c * _LANES, _LANES)],
                outx_hbm.at[pl.ds(c * S + s0, _LANES)], sem0)
            cp.start()
            ocps.append(cp)
        for cp in ocps:
            cp.wait()
        pltpu.sync_copy(valv, outs_hbm.at[pl.ds(s0, _LANES)])

    return run


def kernel(X, samples, num_samples):
    S, N = samples.shape[0], samples.shape[1]
    D = X.shape[-1]
    obj_flat = jnp.transpose(samples, (2, 1, 0)).reshape(-1)
    pmax, pidx = _build_partial(S, N)(obj_flat)
    x_cmaj, score = _build_merge(S, N, D)(pmax, pidx, X.T.reshape(-1))
    return x_cmaj.reshape(D, S).T, score.reshape(S, 1)


# speedup vs baseline: 1.0181x; 1.0002x over previous
import functools

import jax
import jax.numpy as jnp
from jax import lax
from jax.experimental import pallas as pl
from jax.experimental.pallas import tpu as pltpu
from jax.experimental.pallas import tpu_sc as plsc

_NC = 2
_NS = 16
_W = _NC * _NS
_LANES = 16
_CHROWS = 25
_NBUF = 5
_NEG = float("-inf")


def _build_partial(S, N):
    NPW = N // _W
    NCH = NPW // _CHROWS
    NV = S // _LANES
    QUAD = NV // 4
    CH = _CHROWS * S
    assert N % (_W * _CHROWS) == 0 and S % (4 * _LANES) == 0
    assert NCH % _NBUF == 0

    mesh = plsc.VectorSubcoreMesh(
        core_axis_name="c", subcore_axis_name="s",
        num_cores=_NC, num_subcores=_NS)

    @functools.partial(
        pl.kernel,
        out_type=(jax.ShapeDtypeStruct((_W * S,), jnp.float32),
                  jax.ShapeDtypeStruct((_W * S,), jnp.int32)),
        mesh=mesh,
        scratch_types=(
            [pltpu.VMEM((CH,), jnp.float32) for _ in range(_NBUF)] + [
                pltpu.VMEM((S,), jnp.float32),
                pltpu.VMEM((S,), jnp.int32),
            ] + [pltpu.SemaphoreType.DMA for _ in range(_NBUF)]
        ),
        compiler_params=pltpu.CompilerParams(use_tc_tiling_on_sc=False),
    )
    def run(obj_hbm, pmax_hbm, pidx_hbm, *rest):
        bufs = rest[:_NBUF]
        pmax, pidx = rest[_NBUF], rest[_NBUF + 1]
        sems = rest[_NBUF + 2:]
        w = lax.axis_index("s") * _NC + lax.axis_index("c")
        n0 = w * NPW

        def copy_chunk(ch, par):
            return pltpu.make_async_copy(
                obj_hbm.at[pl.ds((n0 + ch * _CHROWS) * S, CH)],
                bufs[par], sems[par])

        IN_FLIGHT = _NBUF - 1

        def consume(buf, ch):
            for q in range(4):
                ms = [pmax[pl.ds((q * QUAD + u) * _LANES, _LANES)]
                      for u in range(QUAD)]
                mis = [pidx[pl.ds((q * QUAD + u) * _LANES, _LANES)]
                       for u in range(QUAD)]

                def body(j, carry, q=q, buf=buf):
                    m8, mi8 = carry
                    m8, mi8 = list(m8), list(mi8)
                    nsplat = jnp.broadcast_to(
                        (n0 + ch * _CHROWS + j).astype(jnp.int32), (_LANES,))
                    for u in range(QUAD):
                        v = buf[pl.ds(j * S + (q * QUAD + u) * _LANES,
                                      _LANES)]
                        cmp = v > m8[u]
                        m8[u] = jnp.where(cmp, v, m8[u])
                        mi8[u] = jnp.where(cmp, nsplat, mi8[u])
                    return tuple(m8), tuple(mi8)

                ms, mis = lax.fori_loop(
                    0, _CHROWS, body, (tuple(ms), tuple(mis)))
                for u in range(QUAD):
                    pmax[pl.ds((q * QUAD + u) * _LANES, _LANES)] = ms[u]
                    pidx[pl.ds((q * QUAD + u) * _LANES, _LANES)] = mis[u]

        neg = jnp.full((_LANES,), _NEG, jnp.float32)
        zero = jnp.zeros((_LANES,), jnp.int32)
        for u in range(NV):
            pmax[pl.ds(u * _LANES, _LANES)] = neg
            pidx[pl.ds(u * _LANES, _LANES)] = zero

        for b in range(IN_FLIGHT):
            copy_chunk(b, b).start()

        def ring(p, carry):
            for b in range(_NBUF):
                ch = p * _NBUF + b

                @pl.when(ch + IN_FLIGHT < NCH)
                def _(ch=ch, b=b):
                    copy_chunk(ch + IN_FLIGHT, (b + IN_FLIGHT) % _NBUF).start()

                copy_chunk(ch, b).wait()
                consume(bufs[b], ch)
            return carry

        lax.fori_loop(0, NCH // _NBUF, ring, 0)

        pltpu.sync_copy(pmax, pmax_hbm.at[pl.ds(w * S, S)])
        pltpu.sync_copy(pidx, pidx_hbm.at[pl.ds(w * S, S)])

    return run


def _build_merge(S, N, D):
    mesh = plsc.VectorSubcoreMesh(
        core_axis_name="c", subcore_axis_name="s",
        num_cores=_NC, num_subcores=_NS)

    @functools.partial(
        pl.kernel,
        out_type=(jax.ShapeDtypeStruct((D * S,), jnp.float32),
                  jax.ShapeDtypeStruct((S,), jnp.float32)),
        mesh=mesh,
        scratch_types=[
            pltpu.VMEM((_W, _LANES), jnp.float32),
            pltpu.VMEM((_W, _LANES), jnp.int32),
            pltpu.VMEM((D * _LANES,), jnp.int32),
            pltpu.VMEM((D * _LANES,), jnp.float32),
            pltpu.VMEM((_LANES,), jnp.float32),
            pltpu.SemaphoreType.DMA,
            pltpu.SemaphoreType.DMA,
            pltpu.SemaphoreType.DMA,
        ],
        compiler_params=pltpu.CompilerParams(use_tc_tiling_on_sc=False),
    )
    def run(pmax_hbm, pidx_hbm, xf_hbm, outx_hbm, outs_hbm,
            mgm, mgi, idxq, gbuf, valv, sem0, sem1, gsem):
        w = lax.axis_index("s") * _NC + lax.axis_index("c")
        s0 = w * _LANES

        cps = []
        for k in range(_W):
            cm = pltpu.make_async_copy(
                pmax_hbm.at[pl.ds(k * S + s0, _LANES)], mgm.at[k], sem0)
            ci = pltpu.make_async_copy(
                pidx_hbm.at[pl.ds(k * S + s0, _LANES)], mgi.at[k], sem1)
            cm.start()
            ci.start()
            cps.append((cm, ci))
        for cm, ci in cps:
            cm.wait()
            ci.wait()

        m = jnp.full((_LANES,), _NEG, jnp.float32)
        mi = jnp.zeros((_LANES,), jnp.int32)
        for k in range(_W):
            v = mgm[k]
            vi = mgi[k]
            cmp = v > m
            m = jnp.where(cmp, v, m)
            mi = jnp.where(cmp, vi, mi)

        valv[...] = m
        for c in range(D):
            idxq[pl.ds(c * _LANES, _LANES)] = mi + c * N
        pltpu.async_copy(xf_hbm.at[idxq], gbuf, gsem).wait()
        ocps = []
        for c in range(D):
            cp = pltpu.make_async_copy(
                gbuf.at[pl.ds(c * _LANES, _LANES)],
                outx_hbm.at[pl.ds(c * S + s0, _LANES)], sem0)
            cp.start()
            ocps.append(cp)
        for cp in ocps:
            cp.wait()
        pltpu.sync_copy(valv, outs_hbm.at[pl.ds(s0, _LANES)])

    return run


def kernel(X, samples, num_samples):
    S, N = samples.shape[0], samples.shape[1]
    D = X.shape[-1]
    obj_flat = jnp.transpose(samples, (2, 1, 0)).reshape(-1)
    pmax, pidx = _build_partial(S, N)(obj_flat)
    x_cmaj, score = _build_merge(S, N, D)(pmax, pidx, X.T.reshape(-1))
    return x_cmaj.reshape(D, S).T, score.reshape(S, 1)
